# baseline (speedup 1.0000x reference)
import jax
import jax.numpy as jnp
from jax.experimental import pallas as pl
from jax.experimental.pallas import tpu as pltpu

B, L, T, KH, DH = 16, 2, 2048, 8, 64
ROW = 2 * KH * DH
NP = B * L * T


def _body(layer_ref, idx_ref, kv_ref, buf_any, out_any, sem):
    del buf_any
    layer = layer_ref[0]
    idx = idx_ref[0]
    for b in range(B):
        pltpu.make_async_copy(
            kv_ref.at[b], out_any.at[(b * L + layer) * T + idx], sem
        ).start()
    for b in range(B):
        pltpu.make_async_copy(
            kv_ref.at[b], out_any.at[(b * L + layer) * T + idx], sem
        ).wait()


@jax.jit
def _run(layer_s, idx_s, kv, buf2):
    return pl.pallas_call(
        _body,
        in_specs=[
            pl.BlockSpec(memory_space=pltpu.SMEM),
            pl.BlockSpec(memory_space=pltpu.SMEM),
            pl.BlockSpec(memory_space=pltpu.VMEM),
            pl.BlockSpec(memory_space=pl.ANY),
        ],
        out_specs=pl.BlockSpec(memory_space=pl.ANY),
        out_shape=jax.ShapeDtypeStruct((NP, ROW), jnp.float32),
        scratch_shapes=[pltpu.SemaphoreType.DMA],
        input_output_aliases={3: 0},
    )(layer_s, idx_s, kv, buf2)


def kernel(buf, k_step, v_step, layer, idx):
    layer = jnp.clip(jnp.asarray(layer, jnp.int32), 0, L - 1)
    idx = jnp.clip(jnp.asarray(idx, jnp.int32), 0, T - 1)
    kv = jnp.concatenate(
        [k_step.reshape(B, KH * DH), v_step.reshape(B, KH * DH)], axis=1
    )
    out2 = _run(layer.reshape(1), idx.reshape(1), kv, buf.reshape(NP, ROW))
    return out2.reshape(B, L, T, 2, KH, DH)
